# baseline (speedup 1.0000x reference)
import functools

import jax
import jax.numpy as jnp
from jax import lax
from jax.experimental import pallas as pl
from jax.experimental.pallas import tpu as pltpu
from jax.experimental.pallas import tpu_sc as plsc

B = 4096
H = 200
L = 51
V = 1000000
U = 100000
D = 64

NC = 2
NS = 16
NW = NC * NS

N = B * H
PER_W = N // NW
GW = 128
CH = 1024
K = CH // GW
NCHUNK = PER_W // CH


def _gather_body(idx_hbm, table_hbm, out_hbm, idx_v, rows_v, sem):
    wid = lax.axis_index("s") * NC + lax.axis_index("c")
    base = wid * PER_W

    def chunk(i, carry):
        off = pl.multiple_of(base + i * CH, CH)
        pltpu.sync_copy(idx_hbm.at[pl.ds(pl.multiple_of(off // GW, K), K)], idx_v)
        handles = [
            pltpu.async_copy(
                table_hbm.at[idx_v.at[j]],
                rows_v.at[pl.ds(j * GW, GW)],
                sem,
            )
            for j in range(K)
        ]
        for h in handles:
            h.wait()
        pltpu.sync_copy(rows_v, out_hbm.at[pl.ds(off, CH)])
        return carry

    lax.fori_loop(0, NCHUNK, chunk, 0)


_gather = functools.partial(
    pl.kernel,
    out_type=jax.ShapeDtypeStruct((N, D), jnp.float32),
    mesh=plsc.VectorSubcoreMesh(core_axis_name="c", subcore_axis_name="s"),
    scratch_types=[
        pltpu.VMEM((K, GW), jnp.int32),
        pltpu.VMEM((CH, D), jnp.float32),
        pltpu.SemaphoreType.DMA,
    ],
    compiler_params=pltpu.CompilerParams(use_tc_tiling_on_sc=False),
)(_gather_body)


BB = 32


def _times_body(row1_ref, tpad_ref, w_ref, b_ref, exp_ref, times_ref, sep_ref):
    t = jnp.log((tpad_ref[...] - row1_ref[...]) + 1.0)
    w = w_ref[...].reshape(1, 1, D)
    bias = b_ref[...].reshape(1, 1, D)
    times_ref[...] = t[:, :, None] * w + bias
    sep_ref[...] = exp_ref[...]


_times = pl.pallas_call(
    _times_body,
    grid=(B // BB,),
    in_specs=[
        pl.BlockSpec((BB, H), lambda i: (i, 0)),
        pl.BlockSpec((BB, 1), lambda i: (i, 0)),
        pl.BlockSpec((1, D), lambda i: (0, 0)),
        pl.BlockSpec((1, D), lambda i: (0, 0)),
        pl.BlockSpec((1, D), lambda i: (0, 0)),
    ],
    out_specs=[
        pl.BlockSpec((BB, H, D), lambda i: (i, 0, 0)),
        pl.BlockSpec((1, D), lambda i: (0, 0)),
    ],
    out_shape=[
        jax.ShapeDtypeStruct((B, H, D), jnp.float32),
        jax.ShapeDtypeStruct((1, D), jnp.float32),
    ],
)


def kernel(row0, row1, row2, row3, row4, row5, row6, row7,
           click_emb, exposure_emb, uid_emb, fc_w, fc_b):
    idx = row0.astype(jnp.int32).reshape(N // GW, GW)
    high_items_emb = _gather(idx, click_emb).reshape(B, H, D)
    tpad = row6[:, L - 1:L]
    times, sep = _times(row1, tpad, fc_w, fc_b.reshape(1, D),
                        exposure_emb[0:1])
    return (high_items_emb, times, sep.reshape(D))


# speedup vs baseline: 1.2286x; 1.2286x over previous
import functools

import jax
import jax.numpy as jnp
from jax import lax
from jax.experimental import pallas as pl
from jax.experimental.pallas import tpu as pltpu
from jax.experimental.pallas import tpu_sc as plsc

B = 4096
H = 200
L = 51
V = 1000000
U = 100000
D = 64

NC = 2
NS = 16
NW = NC * NS

N = B * H
PER_W = N // NW
GW = 128
CH = 1024
K = CH // GW
NCHUNK = PER_W // CH


def _gather_body(idx_hbm, table_hbm, out_hbm, idx_v, rows_v, sem):
    wid = lax.axis_index("s") * NC + lax.axis_index("c")
    base = wid * PER_W

    def chunk(i, carry):
        off = pl.multiple_of(base + i * CH, CH)
        pltpu.sync_copy(idx_hbm.at[pl.ds(pl.multiple_of(off // GW, K), K)], idx_v)
        handles = [
            pltpu.async_copy(
                table_hbm.at[idx_v.at[j]],
                rows_v.at[pl.ds(j * GW, GW)],
                sem,
            )
            for j in range(K)
        ]
        for h in handles:
            h.wait()
        pltpu.sync_copy(rows_v, out_hbm.at[pl.ds(off, CH), pl.ds(0, D)])
        return carry

    lax.fori_loop(0, NCHUNK, chunk, 0)


_gather = functools.partial(
    pl.kernel,
    out_type=jax.ShapeDtypeStruct((N, 2 * D), jnp.float32),
    mesh=plsc.VectorSubcoreMesh(core_axis_name="c", subcore_axis_name="s"),
    scratch_types=[
        pltpu.VMEM((K, GW), jnp.int32),
        pltpu.VMEM((CH, D), jnp.float32),
        pltpu.SemaphoreType.DMA,
    ],
    compiler_params=pltpu.CompilerParams(use_tc_tiling_on_sc=False),
)(_gather_body)


BB = 32


def _times_body(row1_ref, tpad_ref, w_ref, b_ref, exp_ref, times_ref, sep_ref):
    t = jnp.log((tpad_ref[...] - row1_ref[...]) + 1.0)
    w = w_ref[...].reshape(1, 1, D)
    bias = b_ref[...].reshape(1, 1, D)
    times_ref[...] = t[:, :, None] * w + bias
    sep_ref[...] = exp_ref[...]


_times = pl.pallas_call(
    _times_body,
    grid=(B // BB,),
    in_specs=[
        pl.BlockSpec((BB, H), lambda i: (i, 0)),
        pl.BlockSpec((BB, 1), lambda i: (i, 0)),
        pl.BlockSpec((1, D), lambda i: (0, 0)),
        pl.BlockSpec((1, D), lambda i: (0, 0)),
        pl.BlockSpec((1, D), lambda i: (0, 0)),
    ],
    out_specs=[
        pl.BlockSpec((BB, H, D), lambda i: (i, 0, 0)),
        pl.BlockSpec((1, D), lambda i: (0, 0)),
    ],
    out_shape=[
        jax.ShapeDtypeStruct((B, H, D), jnp.float32),
        jax.ShapeDtypeStruct((1, D), jnp.float32),
    ],
)


def kernel(row0, row1, row2, row3, row4, row5, row6, row7,
           click_emb, exposure_emb, uid_emb, fc_w, fc_b):
    idx = row0.astype(jnp.int32).reshape(N // GW, GW)
    high_items_emb = _gather(idx, click_emb)[:, :D].reshape(B, H, D)
    tpad = row6[:, L - 1:L]
    times, sep = _times(row1, tpad, fc_w, fc_b.reshape(1, D),
                        exposure_emb[0:1])
    return (high_items_emb, times, sep.reshape(D))


# speedup vs baseline: 1.6738x; 1.3623x over previous
import functools

import jax
import jax.numpy as jnp
from jax import lax
from jax.experimental import pallas as pl
from jax.experimental.pallas import tpu as pltpu
from jax.experimental.pallas import tpu_sc as plsc

B = 4096
H = 200
L = 51
V = 1000000
U = 100000
D = 64

NC = 2
NS = 16
NW = NC * NS

N = B * H
PER_W = N // NW
GW = 128
CH = 1024
K = CH // GW
NCHUNK = PER_W // CH


def _gather_body(idx_hbm, table_hbm, out_hbm, idx_v, rows_v, sem):
    wid = lax.axis_index("s") * NC + lax.axis_index("c")
    base = wid * PER_W

    def chunk(i, carry):
        off = pl.multiple_of(base + i * CH, CH)
        pltpu.sync_copy(idx_hbm.at[pl.ds(pl.multiple_of(off // GW, K), K)], idx_v)
        handles = [
            pltpu.async_copy(
                table_hbm.at[idx_v.at[j]],
                rows_v.at[pl.ds(j * GW, GW)],
                sem,
            )
            for j in range(K)
        ]
        for h in handles:
            h.wait()
        pltpu.sync_copy(rows_v, out_hbm.at[pl.ds(off, CH), pl.ds(0, D)])
        return carry

    lax.fori_loop(0, NCHUNK, chunk, 0)


_gather = functools.partial(
    pl.kernel,
    out_type=jax.ShapeDtypeStruct((N, 2 * D), jnp.float32),
    mesh=plsc.VectorSubcoreMesh(core_axis_name="c", subcore_axis_name="s"),
    scratch_types=[
        pltpu.VMEM((K, GW), jnp.int32),
        pltpu.VMEM((CH, D), jnp.float32),
        pltpu.SemaphoreType.DMA,
    ],
    compiler_params=pltpu.CompilerParams(use_tc_tiling_on_sc=False),
)(_gather_body)


HB = 8


def _times_body(r1t_ref, tpad_ref, wt_ref, bt_ref, exp_ref, times_ref, sep_ref):
    t = jnp.log((tpad_ref[...] - r1t_ref[...]) + 1.0)
    times_ref[...] = (t[:, None, :] * wt_ref[...][None, :, :]
                      + bt_ref[...][None, :, :])
    sep_ref[...] = exp_ref[...]


_times = pl.pallas_call(
    _times_body,
    grid=(H // HB,),
    in_specs=[
        pl.BlockSpec((HB, B), lambda i: (i, 0)),
        pl.BlockSpec((1, B), lambda i: (0, 0)),
        pl.BlockSpec((D, 1), lambda i: (0, 0)),
        pl.BlockSpec((D, 1), lambda i: (0, 0)),
        pl.BlockSpec((1, D), lambda i: (0, 0)),
    ],
    out_specs=[
        pl.BlockSpec((HB, D, B), lambda i: (i, 0, 0)),
        pl.BlockSpec((1, D), lambda i: (0, 0)),
    ],
    out_shape=[
        jax.ShapeDtypeStruct((H, D, B), jnp.float32),
        jax.ShapeDtypeStruct((1, D), jnp.float32),
    ],
)


def kernel(row0, row1, row2, row3, row4, row5, row6, row7,
           click_emb, exposure_emb, uid_emb, fc_w, fc_b):
    idx = row0.astype(jnp.int32).reshape(N // GW, GW)
    high_items_emb = _gather(idx, click_emb)[:, :D].reshape(B, H, D)
    r1t = row1.T
    tpad_t = row6.T[L - 1:L, :]
    times_t, sep = _times(r1t, tpad_t, fc_w.reshape(D, 1),
                          fc_b.reshape(D, 1), exposure_emb[0:1])
    times = times_t.transpose(2, 0, 1)
    return (high_items_emb, times, sep.reshape(D))


# speedup vs baseline: 1.7100x; 1.0216x over previous
import functools

import jax
import jax.numpy as jnp
from jax import lax
from jax.experimental import pallas as pl
from jax.experimental.pallas import tpu as pltpu
from jax.experimental.pallas import tpu_sc as plsc

B = 4096
H = 200
L = 51
V = 1000000
U = 100000
D = 64

NC = 2
NS = 16
NW = NC * NS

N = B * H
PER_W = N // NW
GW = 128
CH = 512
KC = CH // GW
NCHUNK = PER_W // CH
IDXR = PER_W // GW


def _gather_body(idx_hbm, table_hbm, out_hbm,
                 idx_all, rows0, rows1, g0, g1, w0, w1):
    wid = lax.axis_index("s") * NC + lax.axis_index("c")
    base = pl.multiple_of(wid * PER_W, PER_W)
    pltpu.sync_copy(
        idx_hbm.at[pl.ds(pl.multiple_of(wid * IDXR, IDXR), IDXR)], idx_all)
    rows = (rows0, rows1)
    gsem = (g0, g1)
    wsem = (w0, w1)

    def pair(p, carry):
        for b in range(2):
            c = 2 * p + b
            off = pl.multiple_of(base + c * CH, CH)

            @pl.when(c >= 2)
            def _():
                pltpu.make_async_copy(
                    rows[b], out_hbm.at[pl.ds(off, CH), pl.ds(0, D)],
                    wsem[b]).wait()

            for j in range(KC):
                pltpu.async_copy(
                    table_hbm.at[idx_all.at[c * KC + j]],
                    rows[b].at[pl.ds(j * GW, GW)], gsem[b])
            for j in range(KC):
                pltpu.make_async_copy(
                    table_hbm.at[idx_all.at[c * KC + j]],
                    rows[b].at[pl.ds(j * GW, GW)], gsem[b]).wait()
            pltpu.async_copy(
                rows[b], out_hbm.at[pl.ds(off, CH), pl.ds(0, D)], wsem[b])
        return carry

    lax.fori_loop(0, NCHUNK // 2, pair, 0)
    for b in range(2):
        off = pl.multiple_of(base + (NCHUNK - 2 + b) * CH, CH)
        pltpu.make_async_copy(
            rows[b], out_hbm.at[pl.ds(off, CH), pl.ds(0, D)], wsem[b]).wait()


_gather = functools.partial(
    pl.kernel,
    out_type=jax.ShapeDtypeStruct((N, 2 * D), jnp.float32),
    mesh=plsc.VectorSubcoreMesh(core_axis_name="c", subcore_axis_name="s"),
    scratch_types=[
        pltpu.VMEM((IDXR, GW), jnp.int32),
        pltpu.VMEM((CH, D), jnp.float32),
        pltpu.VMEM((CH, D), jnp.float32),
        pltpu.SemaphoreType.DMA,
        pltpu.SemaphoreType.DMA,
        pltpu.SemaphoreType.DMA,
        pltpu.SemaphoreType.DMA,
    ],
    compiler_params=pltpu.CompilerParams(use_tc_tiling_on_sc=False),
)(_gather_body)


HB = 8


def _times_body(r1t_ref, tpad_ref, wt_ref, bt_ref, exp_ref, times_ref, sep_ref):
    t = jnp.log((tpad_ref[...] - r1t_ref[...]) + 1.0)
    times_ref[...] = (t[:, None, :] * wt_ref[...][None, :, :]
                      + bt_ref[...][None, :, :])
    sep_ref[...] = exp_ref[...]


_times = pl.pallas_call(
    _times_body,
    grid=(H // HB,),
    in_specs=[
        pl.BlockSpec((HB, B), lambda i: (i, 0)),
        pl.BlockSpec((1, B), lambda i: (0, 0)),
        pl.BlockSpec((D, 1), lambda i: (0, 0)),
        pl.BlockSpec((D, 1), lambda i: (0, 0)),
        pl.BlockSpec((1, D), lambda i: (0, 0)),
    ],
    out_specs=[
        pl.BlockSpec((HB, D, B), lambda i: (i, 0, 0)),
        pl.BlockSpec((1, D), lambda i: (0, 0)),
    ],
    out_shape=[
        jax.ShapeDtypeStruct((H, D, B), jnp.float32),
        jax.ShapeDtypeStruct((1, D), jnp.float32),
    ],
)


def kernel(row0, row1, row2, row3, row4, row5, row6, row7,
           click_emb, exposure_emb, uid_emb, fc_w, fc_b):
    idx = row0.astype(jnp.int32).reshape(N // GW, GW)
    high_items_emb = _gather(idx, click_emb)[:, :D].reshape(B, H, D)
    r1t = row1.T
    tpad_t = row6.T[L - 1:L, :]
    times_t, sep = _times(r1t, tpad_t, fc_w.reshape(D, 1),
                          fc_b.reshape(D, 1), exposure_emb[0:1])
    times = times_t.transpose(2, 0, 1)
    return (high_items_emb, times, sep.reshape(D))
